# Initial kernel scaffold; baseline (speedup 1.0000x reference)
import jax
import jax.numpy as jnp
from jax.experimental import pallas as pl


def kernel(classifications, regressions, anchors, annotations):
    raise NotImplementedError("write your pallas kernel here")


# speedup vs baseline: 11.0967x; 11.0967x over previous
import jax
import jax.numpy as jnp
from jax import lax
from jax.experimental import pallas as pl
from jax.experimental.pallas import tpu as pltpu

_ALPHA = 0.25
_TA = 4800


def _body(cls_ref, reg_ref, anc_ref, ann_ref, m_ref, out_ref):
    j = pl.program_id(1)
    m = m_ref[0, 0]

    ay1 = anc_ref[0, 0:1, :]
    ax1 = anc_ref[0, 1:2, :]
    ay2 = anc_ref[0, 2:3, :]
    ax2 = anc_ref[0, 3:4, :]
    nay1 = ay1 / m
    nax1 = ax1 / m
    nay2 = ay2 / m
    nax2 = ax2 / m
    area_a = (nay2 - nay1) * (nax2 - nax1)

    ann = ann_ref[0]

    s_iota = lax.broadcasted_iota(jnp.int32, (8, 1), 0).astype(jnp.float32)
    best = jnp.full((8, _TA), -1.0, jnp.float32)
    idx8 = jnp.zeros((8, _TA), jnp.float32)
    gx1_8 = jnp.zeros((8, _TA), jnp.float32)
    gy1_8 = jnp.zeros((8, _TA), jnp.float32)
    gx2_8 = jnp.zeros((8, _TA), jnp.float32)
    gy2_8 = jnp.zeros((8, _TA), jnp.float32)
    cls8 = jnp.zeros((8, _TA), jnp.float32)
    for g in range(4):
        bx1 = ann[g * 8:(g + 1) * 8, 0:1]
        by1 = ann[g * 8:(g + 1) * 8, 1:2]
        bx2 = ann[g * 8:(g + 1) * 8, 2:3]
        by2 = ann[g * 8:(g + 1) * 8, 3:4]
        bcl = ann[g * 8:(g + 1) * 8, 4:5]
        nbx1 = bx1 / m
        nby1 = by1 / m
        nbx2 = bx2 / m
        nby2 = by2 / m
        area_b = (nbx2 - nbx1) * (nby2 - nby1)
        iw = jnp.minimum(nax2, nbx2) - jnp.maximum(nax1, nbx1)
        ih = jnp.minimum(nay2, nby2) - jnp.maximum(nay1, nby1)
        iw = jnp.maximum(iw, 0.0)
        ih = jnp.maximum(ih, 0.0)
        inter = iw * ih
        ua = jnp.maximum(area_a + area_b - inter, 1e-8)
        iou = inter / ua
        upd = iou > best
        best = jnp.where(upd, iou, best)
        idx8 = jnp.where(upd, g * 8.0 + s_iota, idx8)
        gx1_8 = jnp.where(upd, bx1, gx1_8)
        gy1_8 = jnp.where(upd, by1, gy1_8)
        gx2_8 = jnp.where(upd, bx2, gx2_8)
        gy2_8 = jnp.where(upd, by2, gy2_8)
        cls8 = jnp.where(upd, bcl, cls8)

    iou_max = jnp.max(best, axis=0, keepdims=True)
    at_max = best == iou_max
    idx = jnp.min(jnp.where(at_max, idx8, 1e9), axis=0, keepdims=True)
    win = idx8 == idx
    gx1 = jnp.sum(jnp.where(win, gx1_8, 0.0), axis=0, keepdims=True)
    gy1 = jnp.sum(jnp.where(win, gy1_8, 0.0), axis=0, keepdims=True)
    gx2 = jnp.sum(jnp.where(win, gx2_8, 0.0), axis=0, keepdims=True)
    gy2 = jnp.sum(jnp.where(win, gy2_8, 0.0), axis=0, keepdims=True)
    clsw = jnp.sum(jnp.where(win, cls8, 0.0), axis=0, keepdims=True)

    pos = iou_max >= 0.5
    posf = jnp.where(pos, 1.0, 0.0)
    validf = jnp.where(pos | (iou_max < 0.4), 1.0, 0.0)

    ty = reg_ref[0, 0, 0:1, :]
    tx = reg_ref[0, 0, 1:2, :]
    th = reg_ref[0, 0, 2:3, :]
    tw = reg_ref[0, 0, 3:4, :]
    aw = ax2 - ax1
    ah = ay2 - ay1
    acx = ax1 + 0.5 * aw
    acy = ay1 + 0.5 * ah
    pcx = tx * aw + acx
    pcy = ty * ah + acy
    pw = jnp.exp(tw) * aw
    ph = jnp.exp(th) * ah
    px1 = jnp.maximum(pcx - 0.5 * pw, 0.0)
    py1 = jnp.maximum(pcy - 0.5 * ph, 0.0)
    px2 = jnp.maximum(pcx + 0.5 * pw, 0.0)
    py2 = jnp.maximum(pcy + 0.5 * ph, 0.0)
    cgx1 = jnp.maximum(gx1, 0.0)
    cgy1 = jnp.maximum(gy1, 0.0)
    cgx2 = jnp.maximum(gx2, 0.0)
    cgy2 = jnp.maximum(gy2, 0.0)
    iw2 = jnp.maximum(jnp.minimum(px2, cgx2) - jnp.maximum(px1, cgx1), 0.0)
    ih2 = jnp.maximum(jnp.minimum(py2, cgy2) - jnp.maximum(py1, cgy1), 0.0)
    inter2 = iw2 * ih2
    area_p = jnp.maximum((px2 - px1) * (py2 - py1), 1e-6)
    area_g = jnp.maximum((cgx2 - cgx1) * (cgy2 - cgy1), 1e-6)
    union = area_p + area_g - inter2
    iou2 = inter2 / (union + 1e-7)
    wc = jnp.maximum(jnp.maximum(px2, cgx2) - jnp.minimum(px1, cgx1), 1e-6)
    hc = jnp.maximum(jnp.maximum(py2, cgy2) - jnp.minimum(py1, cgy1), 1e-6)
    area_c = wc * hc
    giou = jnp.clip(iou2 - (area_c - union) / (area_c + 1e-7), -1.0, 1.0)
    g_part = jnp.sum((1.0 - giou) * posf)
    n_part = jnp.sum(posf)

    csel = jnp.where(pos, clsw, -1.0)
    csel_col = jnp.transpose(csel)
    vmask_col = jnp.transpose(validf)
    p = jnp.clip(cls_ref[0], 0.0005, 1.0 - 0.0005)
    ciota = lax.broadcasted_iota(jnp.int32, p.shape, 1).astype(jnp.float32)
    is_t1 = ciota == csel_col
    lpos = (_ALPHA * ((1.0 - p) * (1.0 - p))) * (-jnp.log(p))
    lneg = ((1.0 - _ALPHA) * (p * p)) * (-jnp.log(1.0 - p))
    loss = jnp.where(is_t1, lpos, lneg) * vmask_col
    c_part = jnp.sum(loss)

    @pl.when(j == 0)
    def _():
        out_ref[0, 0, 0] = 0.0
        out_ref[0, 0, 1] = 0.0
        out_ref[0, 0, 2] = 0.0
        out_ref[0, 0, 3] = 0.0

    out_ref[0, 0, 0] += c_part
    out_ref[0, 0, 1] += n_part
    out_ref[0, 0, 2] += g_part


def kernel(classifications, regressions, anchors, annotations):
    B, A, C = classifications.shape
    M = annotations.shape[1]
    TA = _TA
    NB = A // TA

    m = jnp.max(anchors).reshape(1, 1)
    regs4 = regressions.reshape(B, NB, TA, 4).transpose(0, 1, 3, 2)
    ancT = anchors[0].reshape(NB, TA, 4).transpose(0, 2, 1)

    out = pl.pallas_call(
        _body,
        grid=(B, NB),
        in_specs=[
            pl.BlockSpec((1, TA, C), lambda b, j: (b, j, 0)),
            pl.BlockSpec((1, 1, 4, TA), lambda b, j: (b, j, 0, 0)),
            pl.BlockSpec((1, 4, TA), lambda b, j: (j, 0, 0)),
            pl.BlockSpec((1, M, 5), lambda b, j: (b, 0, 0)),
            pl.BlockSpec(memory_space=pltpu.SMEM),
        ],
        out_specs=pl.BlockSpec((1, 1, 4), lambda b, j: (b, 0, 0),
                               memory_space=pltpu.SMEM),
        out_shape=jax.ShapeDtypeStruct((B, 1, 4), jnp.float32),
        compiler_params=pltpu.CompilerParams(
            dimension_semantics=("parallel", "arbitrary"),
        ),
    )(classifications, regs4, ancT, annotations, m)

    cls_sum = out[:, 0, 0]
    npos = out[:, 0, 1]
    gsum = out[:, 0, 2]
    denom = jnp.maximum(npos, 1.0)
    c = cls_sum / denom
    r = jnp.where(npos > 0.0, gsum / denom, 0.0)
    c_loss = jnp.mean(c)
    r_loss = jnp.mean(r)
    return c_loss + r_loss, c_loss, r_loss
